# baseline (speedup 1.0000x reference)
import functools
import math

import jax
import jax.numpy as jnp
from jax import lax
from jax.experimental import pallas as pl
from jax.experimental.pallas import tpu as pltpu
from jax.experimental.pallas import tpu_sc as plsc

N = 10000
E = 320000
D = 128
C = 1.0
EPS = 1e-15
BALL_EPS = 1e-5

NUM_CORES = 2
NUM_SUBCORES = 16
NUM_WORKERS = NUM_CORES * NUM_SUBCORES
EPW = E // NUM_WORKERS
CH = 64
NFULL = EPW // CH
TL = EPW - NFULL * CH
RPS = 624
REM = N - RPS * NUM_SUBCORES
ZR = 16

_SQRT_C = 1.0

_GATHER_DNUMS = lax.GatherDimensionNumbers(
    offset_dims=(), collapsed_slice_dims=(0,), start_index_map=(0,))


def _artanh(x):
    x = jnp.clip(x, -1.0 + 1e-7, 1.0 - 1e-7)
    return 0.5 * jnp.log((1.0 + x) / (1.0 - x))


def _row_norm(x):
    return jnp.maximum(
        jnp.sqrt(jnp.sum(x * x, axis=-1, keepdims=True)), EPS)


def _proj(x):
    norm = _row_norm(x)
    maxnorm = (1.0 - BALL_EPS) / _SQRT_C
    return jnp.where(norm > maxnorm, x / norm * maxnorm, x)


_CMAX = math.atanh(1.0 - BALL_EPS)


def _pre_body(x_ref, w_ref, out_ref):
    x = x_ref[...]
    w = w_ref[...]
    x_norm = _row_norm(x)
    mx = lax.dot_general(
        x, w, (((1,), (1,)), ((), ())),
        precision=lax.Precision.HIGHEST,
        preferred_element_type=jnp.float32)
    mx_norm = _row_norm(mx)
    a = mx_norm / x_norm * _artanh(_SQRT_C * x_norm)
    h = jnp.minimum(a, _CMAX) * mx / (mx_norm * _SQRT_C)
    lo = lax.bitcast_convert_type(
        h[:, :D // 2].astype(jnp.bfloat16), jnp.uint16).astype(jnp.int32)
    hi = lax.bitcast_convert_type(
        h[:, D // 2:].astype(jnp.bfloat16), jnp.uint16).astype(jnp.int32)
    out_ref[...] = lax.bitwise_or(lo, lax.shift_left(hi, 16))


def _post_body(p_ref, out_ref):
    s = p_ref[0] + p_ref[1]
    ns = _row_norm(s)
    xt = (jnp.minimum(_SQRT_C * ns, _CMAX) / (_SQRT_C * ns)) * jax.nn.relu(s)
    nxt = _row_norm(xt)
    out_ref[...] = _proj(
        jnp.tanh(_SQRT_C * nxt) * xt / (_SQRT_C * nxt))


def _scale_chunk(valb, rowsbf, rowsf, n_groups):
    def _group(g, c2):
        vgroup = valb[pl.ds(g * 16, 16)]
        for i in range(16):
            e = g * 16 + i
            v = lax.gather(
                vgroup, jnp.broadcast_to(i, (16,))[:, None],
                _GATHER_DNUMS, slice_sizes=(1,),
                mode=lax.GatherScatterMode.PROMISE_IN_BOUNDS)
            for j in range(D // 32):
                w = rowsbf[e, pl.ds(j * 16, 16)]
                lo = lax.bitcast_convert_type(
                    lax.shift_left(w, 16), jnp.float32)
                hi = lax.bitcast_convert_type(
                    lax.bitwise_and(w, jnp.int32(-65536)), jnp.float32)
                rowsf[e, pl.ds(j * 16, 16)] = lo * v
                rowsf[e, pl.ds(D // 2 + j * 16, 16)] = hi * v
        return c2

    lax.fori_loop(0, n_groups, _group, 0)


def _spmm_body(hidden, adjh, valh, out,
               colb0, rowb0, valb0, rowsbf0, rowsf0,
               colb1, rowb1, valb1, rowsbf1, rowsf1,
               colb2, rowb2, valb2, rowsbf2, rowsf2,
               colt, rowt, valt, rowstbf, rowstf,
               zbuf, acc, semi0, semi1, semi2,
               semg0, semg1, semg2, sems0, sems1, sems2):
    core = lax.axis_index("c")
    sub = lax.axis_index("s")
    wid = sub * NUM_CORES + core
    colb = (colb0, colb1, colb2)
    rowb = (rowb0, rowb1, rowb2)
    valb = (valb0, valb1, valb2)
    rowsbf = (rowsbf0, rowsbf1, rowsbf2)
    rowsf = (rowsf0, rowsf1, rowsf2)
    semi = (semi0, semi1, semi2)
    semg = (semg0, semg1, semg2)
    sems = (sems0, sems1, sems2)

    def _zrow(r, carry):
        for j in range(D // 16):
            zbuf[r, pl.ds(j * 16, 16)] = jnp.zeros((16,), jnp.float32)
        return carry

    lax.fori_loop(0, ZR, _zrow, 0)

    def _zslab(k, carry):
        pltpu.sync_copy(zbuf, acc.at[pl.ds(sub * RPS + k * ZR, ZR)])
        return carry

    lax.fori_loop(0, RPS // ZR, _zslab, 0)

    @pl.when(sub == 0)
    def _zero_rem():
        pltpu.sync_copy(zbuf.at[pl.ds(0, REM)],
                        acc.at[pl.ds(RPS * NUM_SUBCORES, REM)])

    plsc.subcore_barrier()

    def _idx_base(k):
        return jnp.minimum(wid * EPW + k * CH, E - CH)

    def _issue_idx(k, s):
        base = _idx_base(k)
        pltpu.async_copy(adjh.at[pl.ds(E + base, CH)], colb[s], semi[s])
        pltpu.async_copy(adjh.at[pl.ds(base, CH)], rowb[s], semi[s])
        pltpu.async_copy(valh.at[pl.ds(base, CH)], valb[s], semi[s])

    def _wait_idx(s):
        pltpu.make_async_copy(adjh.at[pl.ds(0, CH)], colb[s], semi[s]).wait()
        pltpu.make_async_copy(adjh.at[pl.ds(0, CH)], rowb[s], semi[s]).wait()
        pltpu.make_async_copy(valh.at[pl.ds(0, CH)], valb[s], semi[s]).wait()

    def _issue_gather(s):
        pltpu.async_copy(hidden.at[colb[s]], rowsbf[s], semg[s])

    def _wait_gather(s):
        pltpu.make_async_copy(hidden.at[colb[s]], rowsbf[s], semg[s]).wait()

    def _issue_scatter(s):
        pltpu.async_copy(rowsf[s], acc.at[rowb[s]], sems[s], add=True)

    def _wait_scatter(s):
        pltpu.make_async_copy(rowsf[s], acc.at[rowb[s]], sems[s]).wait()

    def _steady(k, s, first=False):
        nxt, prv = (s + 1) % 3, (s + 2) % 3
        _wait_idx(nxt)
        _issue_gather(nxt)
        _wait_gather(s)
        _scale_chunk(valb[s], rowsbf[s], rowsf[s], CH // 16)
        _issue_scatter(s)
        if not first:
            _wait_scatter(prv)
        _issue_idx(k + 2, prv)

    _issue_idx(0, 0)
    _issue_idx(1, 1)
    _wait_idx(0)
    _issue_gather(0)
    _steady(0, 0, first=True)
    _steady(1, 1)
    _steady(2, 2)

    def _triple(i, carry):
        k0 = 3 + 3 * i
        for t in range(3):
            _steady(k0 + t, t)
        return carry

    nt = (NFULL - 3) // 3
    lax.fori_loop(0, nt, _triple, 0)
    for kk in range(3 + 3 * nt, NFULL):
        _steady(kk, kk % 3)
    _wait_gather(NFULL % 3)
    _wait_idx((NFULL + 1) % 3)
    _wait_scatter((NFULL - 1) % 3)

    tbase = wid * EPW + NFULL * CH
    pltpu.sync_copy(adjh.at[pl.ds(E + tbase, TL)], colt)
    pltpu.sync_copy(adjh.at[pl.ds(tbase, TL)], rowt)
    pltpu.sync_copy(valh.at[pl.ds(tbase, TL)], valt)
    pltpu.async_copy(hidden.at[colt], rowstbf, semg0).wait()
    _scale_chunk(valt, rowstbf, rowstf, TL // 16)
    pltpu.sync_copy(rowstf, acc.at[rowt], add=True)

    plsc.subcore_barrier()
    pltpu.sync_copy(acc.at[pl.ds(sub * RPS, RPS)],
                    out.at[core, pl.ds(sub * RPS, RPS)])

    @pl.when(sub == 0)
    def _copy_rem():
        pltpu.sync_copy(acc.at[pl.ds(RPS * NUM_SUBCORES, REM)],
                        out.at[core, pl.ds(RPS * NUM_SUBCORES, REM)])


def _make_spmm():
    mesh = plsc.VectorSubcoreMesh(core_axis_name="c", subcore_axis_name="s")
    return pl.kernel(
        _spmm_body,
        out_type=jax.ShapeDtypeStruct((NUM_CORES, N, D), jnp.float32),
        mesh=mesh,
        scratch_types=(
            [pltpu.VMEM((CH,), jnp.int32),
             pltpu.VMEM((CH,), jnp.int32),
             pltpu.VMEM((CH,), jnp.float32),
             pltpu.VMEM((CH, D // 2), jnp.int32),
             pltpu.VMEM((CH, D), jnp.float32)] * 3
            + [pltpu.VMEM((TL,), jnp.int32),
               pltpu.VMEM((TL,), jnp.int32),
               pltpu.VMEM((TL,), jnp.float32),
               pltpu.VMEM((TL, D // 2), jnp.int32),
               pltpu.VMEM((TL, D), jnp.float32),
               pltpu.VMEM((ZR, D), jnp.float32),
               pltpu.VMEM_SHARED((N, D), jnp.float32)]
            + [pltpu.SemaphoreType.DMA] * 9
        ),
        compiler_params=pltpu.CompilerParams(use_tc_tiling_on_sc=False),
    )


_PRE_BM = 1000
_POST_BM = 1000


def kernel(x, adj_indices, adj_values, W):
    x = x.astype(jnp.float32)
    W = W.astype(jnp.float32)
    adj = adj_indices.astype(jnp.int32).reshape(2 * E)
    val = adj_values.astype(jnp.float32)

    hidden_i32 = pl.pallas_call(
        _pre_body,
        grid=(N // _PRE_BM,),
        in_specs=[
            pl.BlockSpec((_PRE_BM, D), lambda i: (i, 0)),
            pl.BlockSpec((D, D), lambda i: (0, 0)),
        ],
        out_specs=pl.BlockSpec((_PRE_BM, D // 2), lambda i: (i, 0)),
        out_shape=jax.ShapeDtypeStruct((N, D // 2), jnp.int32),
    )(x, W)

    partials = _make_spmm()(hidden_i32, adj, val)

    out = pl.pallas_call(
        _post_body,
        grid=(N // _POST_BM,),
        in_specs=[
            pl.BlockSpec((NUM_CORES, _POST_BM, D), lambda i: (0, i, 0)),
        ],
        out_specs=pl.BlockSpec((_POST_BM, D), lambda i: (i, 0)),
        out_shape=jax.ShapeDtypeStruct((N, D), jnp.float32),
    )(partials)
    return out


# speedup vs baseline: 2.1002x; 2.1002x over previous
import functools
import math

import jax
import jax.numpy as jnp
from jax import lax
from jax.experimental import pallas as pl
from jax.experimental.pallas import tpu as pltpu
from jax.experimental.pallas import tpu_sc as plsc

N = 10000
E = 320000
D = 128
C = 1.0
EPS = 1e-15
BALL_EPS = 1e-5

NUM_CORES = 2
NUM_SUBCORES = 16
NUM_WORKERS = NUM_CORES * NUM_SUBCORES
EPW = E // NUM_WORKERS
CH = 112
NFULL = EPW // CH
TL = EPW - NFULL * CH
RPS = 624
REM = N - RPS * NUM_SUBCORES
ZR = 16

_SQRT_C = 1.0

_GATHER_DNUMS = lax.GatherDimensionNumbers(
    offset_dims=(), collapsed_slice_dims=(0,), start_index_map=(0,))


def _artanh(x):
    x = jnp.clip(x, -1.0 + 1e-7, 1.0 - 1e-7)
    return 0.5 * jnp.log((1.0 + x) / (1.0 - x))


def _row_norm(x):
    return jnp.maximum(
        jnp.sqrt(jnp.sum(x * x, axis=-1, keepdims=True)), EPS)


def _proj(x):
    norm = _row_norm(x)
    maxnorm = (1.0 - BALL_EPS) / _SQRT_C
    return jnp.where(norm > maxnorm, x / norm * maxnorm, x)


_CMAX = math.atanh(1.0 - BALL_EPS)


def _pre_body(x_ref, w_ref, out_ref):
    x = x_ref[...]
    w = w_ref[...]
    x_norm = _row_norm(x)
    mx = lax.dot_general(
        x, w, (((1,), (1,)), ((), ())),
        precision=lax.Precision.HIGHEST,
        preferred_element_type=jnp.float32)
    mx_norm = _row_norm(mx)
    a = mx_norm / x_norm * _artanh(_SQRT_C * x_norm)
    out_ref[...] = jnp.minimum(a, _CMAX) * mx / (mx_norm * _SQRT_C)


def _post_body(p_ref, out_ref):
    s = p_ref[0] + p_ref[1]
    ns = _row_norm(s)
    xt = (jnp.minimum(_SQRT_C * ns, _CMAX) / (_SQRT_C * ns)) * jax.nn.relu(s)
    nxt = _row_norm(xt)
    out_ref[...] = _proj(
        jnp.tanh(_SQRT_C * nxt) * xt / (_SQRT_C * nxt))


def _scale_chunk(valb, rowsb, n_groups):
    def _group(g, c2):
        vgroup = valb[pl.ds(g * 16, 16)]
        for i in range(16):
            e = g * 16 + i
            v = lax.gather(
                vgroup, jnp.broadcast_to(i, (16,))[:, None],
                _GATHER_DNUMS, slice_sizes=(1,),
                mode=lax.GatherScatterMode.PROMISE_IN_BOUNDS)
            for j in range(D // 16):
                sl = pl.ds(j * 16, 16)
                rowsb[e, sl] = rowsb[e, sl] * v
        return c2

    lax.fori_loop(0, n_groups, _group, 0)


def _spmm_body(hidden, adjh, valh, out,
               colb0, rowb0, valb0, rowsb0,
               colb1, rowb1, valb1, rowsb1,
               colb2, rowb2, valb2, rowsb2,
               colt, rowt, valt, rowst,
               zbuf, acc, semi0, semi1, semi2,
               semg0, semg1, semg2, sems0, sems1, sems2):
    core = lax.axis_index("c")
    sub = lax.axis_index("s")
    wid = sub * NUM_CORES + core
    colb = (colb0, colb1, colb2)
    rowb = (rowb0, rowb1, rowb2)
    valb = (valb0, valb1, valb2)
    rowsb = (rowsb0, rowsb1, rowsb2)
    semi = (semi0, semi1, semi2)
    semg = (semg0, semg1, semg2)
    sems = (sems0, sems1, sems2)

    def _zrow(r, carry):
        for j in range(D // 16):
            zbuf[r, pl.ds(j * 16, 16)] = jnp.zeros((16,), jnp.float32)
        return carry

    lax.fori_loop(0, ZR, _zrow, 0)

    def _zslab(k, carry):
        pltpu.async_copy(zbuf, acc.at[pl.ds(sub * RPS + k * ZR, ZR)], semg0)
        return carry

    lax.fori_loop(0, RPS // ZR, _zslab, 0)

    @pl.when(sub == 0)
    def _zero_rem():
        pltpu.async_copy(zbuf.at[pl.ds(0, REM)],
                         acc.at[pl.ds(RPS * NUM_SUBCORES, REM)], semg0)

    def _zwait(k, carry):
        pltpu.make_async_copy(
            zbuf, acc.at[pl.ds(sub * RPS + k * ZR, ZR)], semg0).wait()
        return carry

    lax.fori_loop(0, RPS // ZR, _zwait, 0)

    @pl.when(sub == 0)
    def _zero_rem_wait():
        pltpu.make_async_copy(
            zbuf.at[pl.ds(0, REM)],
            acc.at[pl.ds(RPS * NUM_SUBCORES, REM)], semg0).wait()

    plsc.subcore_barrier()

    def _idx_base(k):
        return jnp.minimum(wid * EPW + k * CH, E - CH)

    def _issue_idx(k, s):
        base = _idx_base(k)
        pltpu.async_copy(adjh.at[pl.ds(E + base, CH)], colb[s], semi[s])
        pltpu.async_copy(adjh.at[pl.ds(base, CH)], rowb[s], semi[s])
        pltpu.async_copy(valh.at[pl.ds(base, CH)], valb[s], semi[s])

    def _wait_idx(s):
        pltpu.make_async_copy(adjh.at[pl.ds(0, CH)], colb[s], semi[s]).wait()
        pltpu.make_async_copy(adjh.at[pl.ds(0, CH)], rowb[s], semi[s]).wait()
        pltpu.make_async_copy(valh.at[pl.ds(0, CH)], valb[s], semi[s]).wait()

    def _issue_gather(s):
        pltpu.async_copy(hidden.at[colb[s]], rowsb[s], semg[s])

    def _wait_gather(s):
        pltpu.make_async_copy(hidden.at[colb[s]], rowsb[s], semg[s]).wait()

    def _issue_scatter(s):
        pltpu.async_copy(rowsb[s], acc.at[rowb[s]], sems[s], add=True)

    def _wait_scatter(s):
        pltpu.make_async_copy(rowsb[s], acc.at[rowb[s]], sems[s]).wait()

    def _steady(k, s, first=False):
        nxt, prv = (s + 1) % 3, (s + 2) % 3
        _wait_idx(nxt)
        _issue_gather(nxt)
        _wait_gather(s)
        _scale_chunk(valb[s], rowsb[s], CH // 16)
        _issue_scatter(s)
        if not first:
            _wait_scatter(prv)
        _issue_idx(k + 2, prv)

    _issue_idx(0, 0)
    _issue_idx(1, 1)
    _wait_idx(0)
    _issue_gather(0)
    _steady(0, 0, first=True)
    _steady(1, 1)
    _steady(2, 2)

    def _triple(i, carry):
        k0 = 3 + 3 * i
        for t in range(3):
            _steady(k0 + t, t)
        return carry

    nt = (NFULL - 3) // 3
    lax.fori_loop(0, nt, _triple, 0)
    for kk in range(3 + 3 * nt, NFULL):
        _steady(kk, kk % 3)
    _wait_gather(NFULL % 3)
    _wait_idx((NFULL + 1) % 3)
    _wait_scatter((NFULL - 1) % 3)

    tbase = wid * EPW + NFULL * CH
    pltpu.sync_copy(adjh.at[pl.ds(E + tbase, TL)], colt)
    pltpu.sync_copy(adjh.at[pl.ds(tbase, TL)], rowt)
    pltpu.sync_copy(valh.at[pl.ds(tbase, TL)], valt)
    pltpu.async_copy(hidden.at[colt], rowst, semg0).wait()
    _scale_chunk(valt, rowst, TL // 16)
    pltpu.sync_copy(rowst, acc.at[rowt], add=True)

    plsc.subcore_barrier()
    pltpu.sync_copy(acc.at[pl.ds(sub * RPS, RPS)],
                    out.at[core, pl.ds(sub * RPS, RPS)])

    @pl.when(sub == 0)
    def _copy_rem():
        pltpu.sync_copy(acc.at[pl.ds(RPS * NUM_SUBCORES, REM)],
                        out.at[core, pl.ds(RPS * NUM_SUBCORES, REM)])


def _make_spmm():
    mesh = plsc.VectorSubcoreMesh(core_axis_name="c", subcore_axis_name="s")
    return pl.kernel(
        _spmm_body,
        out_type=jax.ShapeDtypeStruct((NUM_CORES, N, D), jnp.float32),
        mesh=mesh,
        scratch_types=(
            [pltpu.VMEM((CH,), jnp.int32),
             pltpu.VMEM((CH,), jnp.int32),
             pltpu.VMEM((CH,), jnp.float32),
             pltpu.VMEM((CH, D), jnp.float32)] * 3
            + [pltpu.VMEM((TL,), jnp.int32),
               pltpu.VMEM((TL,), jnp.int32),
               pltpu.VMEM((TL,), jnp.float32),
               pltpu.VMEM((TL, D), jnp.float32),
               pltpu.VMEM((ZR, D), jnp.float32),
               pltpu.VMEM_SHARED((N, D), jnp.float32)]
            + [pltpu.SemaphoreType.DMA] * 9
        ),
    )


_PRE_BM = 1000
_POST_BM = 1000


def kernel(x, adj_indices, adj_values, W):
    x = x.astype(jnp.float32)
    W = W.astype(jnp.float32)
    adj = adj_indices.astype(jnp.int32).reshape(2 * E)
    val = adj_values.astype(jnp.float32)

    hidden_e = pl.pallas_call(
        _pre_body,
        grid=(N // _PRE_BM,),
        in_specs=[
            pl.BlockSpec((_PRE_BM, D), lambda i: (i, 0)),
            pl.BlockSpec((D, D), lambda i: (0, 0)),
        ],
        out_specs=pl.BlockSpec((_PRE_BM, D), lambda i: (i, 0)),
        out_shape=jax.ShapeDtypeStruct((N, D), jnp.float32),
    )(x, W)

    partials = _make_spmm()(hidden_e, adj, val)

    out = pl.pallas_call(
        _post_body,
        grid=(N // _POST_BM,),
        in_specs=[
            pl.BlockSpec((NUM_CORES, _POST_BM, D), lambda i: (0, i, 0)),
        ],
        out_specs=pl.BlockSpec((_POST_BM, D), lambda i: (i, 0)),
        out_shape=jax.ShapeDtypeStruct((N, D), jnp.float32),
    )(partials)
    return out
